# baseline (speedup 1.0000x reference)
import jax
import jax.numpy as jnp
from jax import lax
from jax.experimental import pallas as pl
from jax.experimental.pallas import tpu as pltpu

_N_WORDS = 100000
_DIM = 16


def _body(ix_ref, vec_in, sum_in, cnt_in, cov_in, sum_out, cnt_out, cov_out):
    ix = ix_ref[0]
    vec = vec_in[...]

    r = ix % 8
    row_i = lax.broadcasted_iota(jnp.int32, (8, _DIM), 0)
    vec_b = jnp.broadcast_to(vec, (8, _DIM))
    sum_out[...] = sum_in[...] + jnp.where(row_i == r, vec_b, 0.0)

    r2 = (ix // 8) % 8
    c2 = ix % 8
    row_c = lax.broadcasted_iota(jnp.int32, (8, 8), 0)
    col_c = lax.broadcasted_iota(jnp.int32, (8, 8), 1)
    hit = jnp.logical_and(row_c == r2, col_c == c2)
    cnt_out[...] = cnt_in[...] + hit.astype(jnp.int32)

    outer = lax.dot_general(vec, vec, (((0,), (0,)), ((), ())),
                            preferred_element_type=jnp.float32)
    cov_out[...] = cov_in[...] + outer.reshape(1, _DIM, _DIM)


def kernel(ix, vec, sum_buf, counts, cov_buf):
    ix_arr = jnp.reshape(jnp.asarray(ix, jnp.int32), (1,))
    cnt2d = counts.reshape(_N_WORDS // 8, 8)
    grid_spec = pltpu.PrefetchScalarGridSpec(
        num_scalar_prefetch=1,
        grid=(1,),
        in_specs=[
            pl.BlockSpec((1, _DIM), lambda i, s: (0, 0)),
            pl.BlockSpec((8, _DIM), lambda i, s: (s[0] // 8, 0)),
            pl.BlockSpec((8, 8), lambda i, s: (s[0] // 64, 0)),
            pl.BlockSpec((1, _DIM, _DIM), lambda i, s: (s[0], 0, 0)),
        ],
        out_specs=[
            pl.BlockSpec((8, _DIM), lambda i, s: (s[0] // 8, 0)),
            pl.BlockSpec((8, 8), lambda i, s: (s[0] // 64, 0)),
            pl.BlockSpec((1, _DIM, _DIM), lambda i, s: (s[0], 0, 0)),
        ],
    )
    out = pl.pallas_call(
        _body,
        grid_spec=grid_spec,
        out_shape=[
            jax.ShapeDtypeStruct((_N_WORDS, _DIM), jnp.float32),
            jax.ShapeDtypeStruct((_N_WORDS // 8, 8), jnp.int32),
            jax.ShapeDtypeStruct((_N_WORDS, _DIM, _DIM), jnp.float32),
        ],
        input_output_aliases={2: 0, 3: 1, 4: 2},
    )(ix_arr, vec.reshape(1, _DIM), sum_buf, cnt2d, cov_buf)
    return out[0], out[1].reshape(_N_WORDS), out[2]


# speedup vs baseline: 3.4627x; 3.4627x over previous
import jax
import jax.numpy as jnp
from jax import lax
from jax.experimental import pallas as pl
from jax.experimental.pallas import tpu as pltpu

_N_WORDS = 100000
_DIM = 16
_CNT_ROWS = 782
_CPAD = _CNT_ROWS * 128 - _N_WORDS


def _body(ix_ref, vec_in, sum_in, cnt_in, cov_in, sum_out, cnt_out, cov_out):
    ix = ix_ref[0]
    vec = vec_in[...]

    r = (ix // 8) % 8
    g = ix % 8
    row_i = lax.broadcasted_iota(jnp.int32, (8, 128), 0)
    lane_i = lax.broadcasted_iota(jnp.int32, (8, 128), 1)
    vec_t8 = jnp.broadcast_to(jnp.concatenate([vec] * 8, axis=1), (8, 128))
    hit_s = jnp.logical_and(row_i == r, lane_i // _DIM == g)
    sum_out[...] = sum_in[...] + jnp.where(hit_s, vec_t8, 0.0)

    r2 = (ix // 128) % 8
    c2 = ix % 128
    hit_c = jnp.logical_and(row_i == r2, lane_i == c2)
    cnt_out[...] = cnt_in[...] + hit_c.astype(jnp.int32)

    iota_j = lax.broadcasted_iota(jnp.int32, (_DIM, 256), 0)
    iota_l = lax.broadcasted_iota(jnp.int32, (_DIM, 256), 1)
    m_int = (iota_l // _DIM == iota_j).astype(jnp.float32)
    b = lax.dot_general(vec, m_int, (((1,), (0,)), ((), ())),
                        precision=lax.Precision.HIGHEST,
                        preferred_element_type=jnp.float32)
    a = jnp.concatenate([vec] * _DIM, axis=1)
    outer_flat = a * b
    r3 = ix % 8
    row_i2 = lax.broadcasted_iota(jnp.int32, (8, 256), 0)
    outer_b = jnp.broadcast_to(outer_flat, (8, 256))
    cov_out[...] = cov_in[...] + jnp.where(row_i2 == r3, outer_b, 0.0)


def kernel(ix, vec, sum_buf, counts, cov_buf):
    ix_arr = jnp.reshape(jnp.asarray(ix, jnp.int32), (1,))
    cpad = jnp.concatenate(
        [counts, jnp.zeros((_CPAD,), jnp.int32)]).reshape(_CNT_ROWS, 128)
    grid_spec = pltpu.PrefetchScalarGridSpec(
        num_scalar_prefetch=1,
        grid=(1,),
        in_specs=[
            pl.BlockSpec((1, _DIM), lambda i, s: (0, 0)),
            pl.BlockSpec((8, 128), lambda i, s: (s[0] // 64, 0)),
            pl.BlockSpec((8, 128), lambda i, s: (s[0] // 1024, 0)),
            pl.BlockSpec((8, 256), lambda i, s: (s[0] // 8, 0)),
        ],
        out_specs=[
            pl.BlockSpec((8, 128), lambda i, s: (s[0] // 64, 0)),
            pl.BlockSpec((8, 128), lambda i, s: (s[0] // 1024, 0)),
            pl.BlockSpec((8, 256), lambda i, s: (s[0] // 8, 0)),
        ],
    )
    out = pl.pallas_call(
        _body,
        grid_spec=grid_spec,
        out_shape=[
            jax.ShapeDtypeStruct((_N_WORDS // 8, 128), jnp.float32),
            jax.ShapeDtypeStruct((_CNT_ROWS, 128), jnp.int32),
            jax.ShapeDtypeStruct((_N_WORDS, 256), jnp.float32),
        ],
        input_output_aliases={2: 0, 3: 1, 4: 2},
    )(ix_arr, vec.reshape(1, _DIM),
      sum_buf.reshape(_N_WORDS // 8, 128), cpad,
      cov_buf.reshape(_N_WORDS, 256))
    return (out[0].reshape(_N_WORDS, _DIM),
            out[1].reshape(-1)[:_N_WORDS],
            out[2].reshape(_N_WORDS, _DIM, _DIM))
